# baseline (speedup 1.0000x reference)
import functools

import jax
import jax.numpy as jnp
from jax.experimental import pallas as pl
from jax.experimental.pallas import tpu as pltpu
from jax.experimental.pallas import tpu_sc as plsc

N = 10000
E = 160000
NUM_GRAPHS = 8
NTYPES = 36
DIMS = [(128, 8), (8, 8), (8, 8), (8, 64)]

NC = 2
NS = 16
NW = NC * NS
CH = 128
CW = 40
EW = CH * CW
EPAD = EW * NW
NA = 10240
DW = 128
GNB = 6
SNB = 2


def _sc_gather_call(table, src2d, dg):
    mesh = plsc.VectorSubcoreMesh(core_axis_name="c", subcore_axis_name="s")

    @functools.partial(
        pl.kernel,
        mesh=mesh,
        out_type=jax.ShapeDtypeStruct((EPAD, dg), jnp.float32),
        scratch_types=[
            pltpu.VMEM((CW, CH), jnp.int32),
            pltpu.VMEM((2, CH, dg), jnp.float32),
            pltpu.SemaphoreType.DMA,
            pltpu.SemaphoreType.DMA,
        ],
    )
    def gk(h_hbm, src_hbm, out_hbm, idx_v, buf_v, gsem, wsem):
        cid = jax.lax.axis_index("c")
        sid = jax.lax.axis_index("s")
        wid = cid * NS + sid
        pltpu.sync_copy(src_hbm.at[pl.ds(wid * CW, CW)], idx_v)

        def start_g(j, slot):
            return pltpu.async_copy(h_hbm.at[idx_v.at[j]], buf_v.at[slot], gsem)

        def start_w(j, slot):
            return pltpu.async_copy(
                buf_v.at[slot], out_hbm.at[pl.ds(wid * EW + j * CH, CH)], wsem)

        gh = {j: start_g(j, j % GNB) for j in range(min(GNB, CW))}
        wh = {}
        for j in range(CW):
            gh[j].wait()
            wh[j] = start_w(j, j % GNB)
            if j >= 2:
                wh[j - 2].wait()
                nxt = j - 2 + GNB
                if nxt < CW:
                    gh[nxt] = start_g(nxt, nxt % GNB)
        wh[CW - 2].wait()
        wh[CW - 1].wait()

    return gk(table, src2d)


def _sc_scatter_call(msg, dst2d, zeros, dm):
    mesh = plsc.VectorSubcoreMesh(core_axis_name="c", subcore_axis_name="s")
    zr = NA // NS

    @functools.partial(
        pl.kernel,
        mesh=mesh,
        out_type=jax.ShapeDtypeStruct((NC, NA, dm), jnp.float32),
        scratch_types=[
            pltpu.VMEM((CW, CH), jnp.int32),
            pltpu.VMEM((SNB, CH, dm), jnp.float32),
            pltpu.VMEM_SHARED((NA, dm), jnp.float32),
            pltpu.SemaphoreType.DMA,
            pltpu.SemaphoreType.DMA,
            pltpu.SemaphoreType.DMA,
        ],
    )
    def sk(msg_hbm, dst_hbm, z_hbm, out_hbm, idx_v, buf_v, acc_sh,
           lsem, asem, osem):
        cid = jax.lax.axis_index("c")
        sid = jax.lax.axis_index("s")
        wid = cid * NS + sid

        pltpu.sync_copy(dst_hbm.at[pl.ds(wid * CW, CW)], idx_v)
        pltpu.sync_copy(z_hbm, acc_sh.at[pl.ds(sid * zr, zr)])
        plsc.subcore_barrier()

        def start_l(j, slot):
            return pltpu.async_copy(
                msg_hbm.at[pl.ds(wid * EW + j * CH, CH)], buf_v.at[slot], lsem)

        def start_a(j, slot):
            return pltpu.async_copy(buf_v.at[slot], acc_sh.at[idx_v.at[j]],
                                    asem, add=True)

        lh = {0: start_l(0, 0)}
        ah = {}
        for j in range(CW):
            lh[j].wait()
            ah[j] = start_a(j, j % SNB)
            if j >= 1:
                ah[j - 1].wait()
            if j + 1 < CW:
                lh[j + 1] = start_l(j + 1, (j + 1) % SNB)
        ah[CW - 1].wait()
        plsc.subcore_barrier()

        dh = [pltpu.async_copy(acc_sh.at[pl.ds(sid * zr + r * CH, CH)],
                               out_hbm.at[cid, pl.ds(sid * zr + r * CH, CH)],
                               osem)
              for r in range(zr // CH)]
        for h in dh:
            h.wait()

    return sk(msg, dst2d, zeros)


def _tc_prep_call(tabs):
    def body(*refs):
        ins, outs = refs[:12], refs[12:]
        for li in range(4):
            e = ins[3 * li][...]
            hb = ins[3 * li + 1]
            gb = ins[3 * li + 2]
            rows = [e * hb[k:k + 1, :] + gb[k:k + 1, :] for k in range(3)]
            outs[li][...] = jnp.concatenate(rows, axis=0)

    flat = [a for t in tabs for a in t]
    out_shape = tuple(
        jax.ShapeDtypeStruct((108, t[0].shape[1]), jnp.float32) for t in tabs)
    return pl.pallas_call(body, out_shape=out_shape)(*flat)


def _tc_msg_call(hj, et, f0, f1, tab, ci, co, dm, be, count_col):
    d = ci * co
    dgin = hj.shape[1]
    grid = EPAD // be

    def body(hj_ref, et_ref, f0_ref, f1_ref, t_ref, out_ref):
        lane = jax.lax.broadcasted_iota(jnp.int32, (be, 3 * NTYPES), 1)
        lt = lane - NTYPES * (lane // NTYPES)
        e = et_ref[...].astype(jnp.int32)
        m = lt == e
        coeff = jnp.where(lane < NTYPES, 1.0,
                          jnp.where(lane < 2 * NTYPES, f0_ref[...], f1_ref[...]))
        p = jnp.where(m, coeff, 0.0)
        arg = jnp.dot(p.astype(jnp.bfloat16), t_ref[...].astype(jnp.bfloat16),
                      preferred_element_type=jnp.float32)
        w = jnp.maximum(arg, 0.0)
        hjc = hj_ref[...][:, :ci]
        ht = jnp.concatenate([hjc] * co, axis=1)
        prod = w * ht
        msg = jnp.sum(prod.reshape(be, co, ci), axis=2)
        if dm > co:
            cols = [msg]
            if count_col:
                cols.append(jnp.ones((be, 1), jnp.float32))
                cols.append(jnp.zeros((be, dm - co - 1), jnp.float32))
            else:
                cols.append(jnp.zeros((be, dm - co), jnp.float32))
            out_ref[...] = jnp.concatenate(cols, axis=1)
        else:
            out_ref[...] = msg

    return pl.pallas_call(
        body,
        grid=(grid,),
        in_specs=[
            pl.BlockSpec((be, dgin), lambda i: (i, 0)),
            pl.BlockSpec((be, 1), lambda i: (i, 0)),
            pl.BlockSpec((be, 1), lambda i: (i, 0)),
            pl.BlockSpec((be, 1), lambda i: (i, 0)),
            pl.BlockSpec((108, d), lambda i: (0, 0)),
        ],
        out_specs=pl.BlockSpec((be, dm), lambda i: (i, 0)),
        out_shape=jax.ShapeDtypeStruct((EPAD, dm), jnp.float32),
    )(hj, et, f0, f1, tab)


def _tc_combine_call(s0, s1, h, cnt, root, bias, ci, co, dout, emit_cnt):
    bn = 1000
    grid = N // bn
    dm = s0.shape[1]
    din = h.shape[1]

    def body(*refs):
        if emit_cnt:
            s0_ref, s1_ref, h_ref, root_ref, bias_ref, out_ref, cnt_ref = refs
        else:
            s0_ref, s1_ref, h_ref, cin_ref, root_ref, bias_ref, out_ref = refs
        p0 = s0_ref[...]
        p1 = s1_ref[...]
        s = p0[:, :co] + p1[:, :co]
        if emit_cnt:
            c = p0[:, co:co + 1] + p1[:, co:co + 1]
        else:
            c = cin_ref[...]
        agg = s / jnp.maximum(c, 1.0)
        hc = h_ref[...][:, :ci]
        o = jnp.maximum(
            agg + jnp.dot(hc, root_ref[...], preferred_element_type=jnp.float32)
            + bias_ref[...], 0.0)
        if dout > co:
            o = jnp.concatenate([o, jnp.zeros((bn, dout - co), jnp.float32)], axis=1)
        out_ref[...] = o
        if emit_cnt:
            cnt_ref[...] = c

    in_specs = [
        pl.BlockSpec((bn, dm), lambda i: (i, 0)),
        pl.BlockSpec((bn, dm), lambda i: (i, 0)),
        pl.BlockSpec((bn, din), lambda i: (i, 0)),
    ]
    args = [s0, s1, h]
    if not emit_cnt:
        in_specs.append(pl.BlockSpec((bn, 1), lambda i: (i, 0)))
        args.append(cnt)
    in_specs += [
        pl.BlockSpec((ci, co), lambda i: (0, 0)),
        pl.BlockSpec((1, co), lambda i: (0, 0)),
    ]
    args += [root, bias]
    if emit_cnt:
        out_specs = (pl.BlockSpec((bn, dout), lambda i: (i, 0)),
                     pl.BlockSpec((bn, 1), lambda i: (i, 0)))
        out_shape = (jax.ShapeDtypeStruct((N, dout), jnp.float32),
                     jax.ShapeDtypeStruct((N, 1), jnp.float32))
    else:
        out_specs = pl.BlockSpec((bn, dout), lambda i: (i, 0))
        out_shape = jax.ShapeDtypeStruct((N, dout), jnp.float32)
    return pl.pallas_call(
        body, grid=(grid,), in_specs=in_specs, out_specs=out_specs,
        out_shape=out_shape)(*args)


def _tc_pool_call(h4, ct, bt, wt, cb):
    def body(h_ref, ct_ref, bt_ref, wt_ref, cb_ref, out_ref):
        h = h_ref[...][:, :64]
        seg = jnp.where(ct_ref[...] == 1, bt_ref[...], -1)
        rows = jax.lax.broadcasted_iota(jnp.int32, (NUM_GRAPHS, N), 0)
        oh = (rows == seg).astype(jnp.float32)
        s = jnp.dot(oh, h, preferred_element_type=jnp.float32)
        cnt = jnp.sum(oh, axis=1, keepdims=True)
        pooled = s / jnp.maximum(cnt, 1.0)
        logits = jnp.dot(pooled, wt_ref[...],
                         preferred_element_type=jnp.float32) + cb_ref[...]
        out_ref[...] = 1.0 / (1.0 + jnp.exp(-logits))

    return pl.pallas_call(
        body,
        out_shape=jax.ShapeDtypeStruct((NUM_GRAPHS, 1), jnp.float32),
    )(h4, ct, bt, wt, cb)


def _relayout(p, ci, co):
    d = ci * co
    emb_r = p['emb'].reshape(NTYPES, ci, co).transpose(0, 2, 1).reshape(NTYPES, d)

    def pv(v):
        return v.reshape(ci, co).T.reshape(1, d)

    hb = jnp.concatenate([pv(p['bh']), pv(p['Wh'][:, 0]), pv(p['Wh'][:, 1])], 0)
    gb = jnp.concatenate([pv(p['bg']), pv(p['Wg'][:, 0]), pv(p['Wg'][:, 1])], 0)
    return emb_r, hb, gb


def kernel(x, edge_index, edge_attr, cell_type, batch, params):
    pad = EPAD - E
    src2d = jnp.concatenate(
        [edge_index[0], jnp.zeros((pad,), jnp.int32)]).reshape(EPAD // CH, CH)
    dst2d = jnp.concatenate(
        [edge_index[1], jnp.full((pad,), N, jnp.int32)]).reshape(EPAD // CH, CH)
    et = jnp.pad(edge_attr[:, 0:1], ((0, pad), (0, 0)))
    f0 = jnp.pad(edge_attr[:, 1:2], ((0, pad), (0, 0)))
    f1 = jnp.pad(edge_attr[:, 2:3], ((0, pad), (0, 0)))

    names = ['conv1', 'conv2', 'conv3', 'conv4']
    tabs = [_relayout(params[n], ci, co) for n, (ci, co) in zip(names, DIMS)]
    ts = _tc_prep_call(tabs)

    bes = [512, 2048, 2048, 512]
    h = x
    cnt = None
    zeros = jnp.zeros((NA // NS, DW), jnp.float32)
    for li, (name, (ci, co)) in enumerate(zip(names, DIMS)):
        p = params[name]
        dm = DW
        hj = _sc_gather_call(h, src2d, DW)
        msg = _tc_msg_call(hj, et, f0, f1, ts[li], ci, co, dm, bes[li],
                           count_col=(li == 0))
        part = _sc_scatter_call(msg, dst2d, zeros, dm)
        bias = p['bias'].reshape(1, co)
        if li == 0:
            h, cnt = _tc_combine_call(part[0], part[1], h, None, p['root'],
                                      bias, ci, co, DW, True)
        else:
            h = _tc_combine_call(part[0], part[1], h, cnt, p['root'], bias,
                                 ci, co, DW, False)

    ct = cell_type.reshape(1, N)
    bt = batch.reshape(1, N)
    return _tc_pool_call(h, ct, bt, params['cls_W'].T,
                         params['cls_b'].reshape(1, 1))


# speedup vs baseline: 2.2130x; 2.2130x over previous
import functools

import jax
import jax.numpy as jnp
from jax.experimental import pallas as pl
from jax.experimental.pallas import tpu as pltpu
from jax.experimental.pallas import tpu_sc as plsc

N = 10000
E = 160000
NUM_GRAPHS = 8
NTYPES = 36
DIMS = [(128, 8), (8, 8), (8, 8), (8, 64)]

NC = 2
NS = 16
NW = NC * NS
CH = 128
CW = 40
EW = CH * CW
EPAD = EW * NW
NA = 10240
DW = 128
GNB = 6
SNB = 2


def _sc_gather_call(table, src2d, dg):
    mesh = plsc.VectorSubcoreMesh(core_axis_name="c", subcore_axis_name="s")

    @functools.partial(
        pl.kernel,
        mesh=mesh,
        out_type=jax.ShapeDtypeStruct((EPAD, dg), jnp.float32),
        scratch_types=[
            pltpu.VMEM((CW, CH), jnp.int32),
            pltpu.VMEM((2, CH, dg), jnp.float32),
            pltpu.SemaphoreType.DMA,
            pltpu.SemaphoreType.DMA,
        ],
    )
    def gk(h_hbm, src_hbm, out_hbm, idx_v, buf_v, gsem, wsem):
        cid = jax.lax.axis_index("c")
        sid = jax.lax.axis_index("s")
        wid = cid * NS + sid
        pltpu.sync_copy(src_hbm.at[pl.ds(wid * CW, CW)], idx_v)

        def start_g(j, slot):
            return pltpu.async_copy(h_hbm.at[idx_v.at[j]], buf_v.at[slot], gsem)

        def start_w(j, slot):
            return pltpu.async_copy(
                buf_v.at[slot], out_hbm.at[pl.ds(wid * EW + j * CH, CH)], wsem)

        gh = {j: start_g(j, j % GNB) for j in range(min(GNB, CW))}
        wh = {}
        for j in range(CW):
            gh[j].wait()
            wh[j] = start_w(j, j % GNB)
            if j >= 2:
                wh[j - 2].wait()
                nxt = j - 2 + GNB
                if nxt < CW:
                    gh[nxt] = start_g(nxt, nxt % GNB)
        wh[CW - 2].wait()
        wh[CW - 1].wait()

    return gk(table, src2d)


def _sc_scatter_call(msg, dst2d, zeros, dm):
    mesh = plsc.VectorSubcoreMesh(core_axis_name="c", subcore_axis_name="s")
    zr = NA // NS

    @functools.partial(
        pl.kernel,
        mesh=mesh,
        out_type=jax.ShapeDtypeStruct((NC, NA, dm), jnp.float32),
        scratch_types=[
            pltpu.VMEM((CW, CH), jnp.int32),
            pltpu.VMEM((SNB, CH, dm), jnp.float32),
            pltpu.VMEM_SHARED((NA, dm), jnp.float32),
            pltpu.SemaphoreType.DMA,
            pltpu.SemaphoreType.DMA,
            pltpu.SemaphoreType.DMA,
        ],
    )
    def sk(msg_hbm, dst_hbm, z_hbm, out_hbm, idx_v, buf_v, acc_sh,
           lsem, asem, osem):
        cid = jax.lax.axis_index("c")
        sid = jax.lax.axis_index("s")
        wid = cid * NS + sid

        pltpu.sync_copy(dst_hbm.at[pl.ds(wid * CW, CW)], idx_v)
        pltpu.sync_copy(z_hbm, acc_sh.at[pl.ds(sid * zr, zr)])
        plsc.subcore_barrier()

        def start_l(j, slot):
            return pltpu.async_copy(
                msg_hbm.at[pl.ds(wid * EW + j * CH, CH)], buf_v.at[slot], lsem)

        def start_a(j, slot):
            return pltpu.async_copy(buf_v.at[slot], acc_sh.at[idx_v.at[j]],
                                    asem, add=True)

        lh = {0: start_l(0, 0)}
        ah = {}
        for j in range(CW):
            lh[j].wait()
            ah[j] = start_a(j, j % SNB)
            if j >= 1:
                ah[j - 1].wait()
            if j + 1 < CW:
                lh[j + 1] = start_l(j + 1, (j + 1) % SNB)
        ah[CW - 1].wait()
        plsc.subcore_barrier()

        dh = [pltpu.async_copy(acc_sh.at[pl.ds(sid * zr + r * CH, CH)],
                               out_hbm.at[cid, pl.ds(sid * zr + r * CH, CH)],
                               osem)
              for r in range(zr // CH)]
        for h in dh:
            h.wait()

    return sk(msg, dst2d, zeros)


def _tc_prep_call(tabs):
    def body(*refs):
        ins, outs = refs[:12], refs[12:]
        for li in range(4):
            e = ins[3 * li][...]
            hb = ins[3 * li + 1]
            gb = ins[3 * li + 2]
            rows = [e * hb[k:k + 1, :] + gb[k:k + 1, :] for k in range(3)]
            outs[li][...] = jnp.concatenate(rows, axis=0)

    flat = [a for t in tabs for a in t]
    out_shape = tuple(
        jax.ShapeDtypeStruct((108, t[0].shape[1]), jnp.float32) for t in tabs)
    return pl.pallas_call(body, out_shape=out_shape)(*flat)


def _tc_msg_call(hj, et, f0, f1, tab, ci, co, dm, be, count_col):
    d = ci * co
    dgin = hj.shape[1]
    grid = EPAD // be

    def body(hj_ref, et_ref, f0_ref, f1_ref, t_ref, out_ref):
        lane = jax.lax.broadcasted_iota(jnp.int32, (be, 3 * NTYPES), 1)
        lt = lane - NTYPES * (lane // NTYPES)
        e = et_ref[...].astype(jnp.int32)
        m = lt == e
        coeff = jnp.where(lane < NTYPES, 1.0,
                          jnp.where(lane < 2 * NTYPES, f0_ref[...], f1_ref[...]))
        p = jnp.where(m, coeff, 0.0)
        arg = jnp.dot(p.astype(jnp.bfloat16), t_ref[...].astype(jnp.bfloat16),
                      preferred_element_type=jnp.float32)
        w = jnp.maximum(arg, 0.0)
        hjc = hj_ref[...][:, :ci]
        ht = jnp.concatenate([hjc] * co, axis=1)
        prod = w * ht
        ko = jax.lax.broadcasted_iota(jnp.int32, (d, co), 0) // ci
        oo = jax.lax.broadcasted_iota(jnp.int32, (d, co), 1)
        red = (ko == oo).astype(jnp.bfloat16)
        msg = jnp.dot(prod.astype(jnp.bfloat16), red,
                      preferred_element_type=jnp.float32)
        if dm > co:
            cols = [msg]
            if count_col:
                cols.append(jnp.ones((be, 1), jnp.float32))
                cols.append(jnp.zeros((be, dm - co - 1), jnp.float32))
            else:
                cols.append(jnp.zeros((be, dm - co), jnp.float32))
            out_ref[...] = jnp.concatenate(cols, axis=1)
        else:
            out_ref[...] = msg

    return pl.pallas_call(
        body,
        grid=(grid,),
        in_specs=[
            pl.BlockSpec((be, dgin), lambda i: (i, 0)),
            pl.BlockSpec((be, 1), lambda i: (i, 0)),
            pl.BlockSpec((be, 1), lambda i: (i, 0)),
            pl.BlockSpec((be, 1), lambda i: (i, 0)),
            pl.BlockSpec((108, d), lambda i: (0, 0)),
        ],
        out_specs=pl.BlockSpec((be, dm), lambda i: (i, 0)),
        out_shape=jax.ShapeDtypeStruct((EPAD, dm), jnp.float32),
    )(hj, et, f0, f1, tab)


def _tc_combine_call(s0, s1, h, cnt, root, bias, ci, co, dout, emit_cnt):
    bn = 1000
    grid = N // bn
    dm = s0.shape[1]
    din = h.shape[1]

    def body(*refs):
        if emit_cnt:
            s0_ref, s1_ref, h_ref, root_ref, bias_ref, out_ref, cnt_ref = refs
        else:
            s0_ref, s1_ref, h_ref, cin_ref, root_ref, bias_ref, out_ref = refs
        p0 = s0_ref[...]
        p1 = s1_ref[...]
        s = p0[:, :co] + p1[:, :co]
        if emit_cnt:
            c = p0[:, co:co + 1] + p1[:, co:co + 1]
        else:
            c = cin_ref[...]
        agg = s / jnp.maximum(c, 1.0)
        hc = h_ref[...][:, :ci]
        o = jnp.maximum(
            agg + jnp.dot(hc, root_ref[...], preferred_element_type=jnp.float32)
            + bias_ref[...], 0.0)
        if dout > co:
            o = jnp.concatenate([o, jnp.zeros((bn, dout - co), jnp.float32)], axis=1)
        out_ref[...] = o
        if emit_cnt:
            cnt_ref[...] = c

    in_specs = [
        pl.BlockSpec((bn, dm), lambda i: (i, 0)),
        pl.BlockSpec((bn, dm), lambda i: (i, 0)),
        pl.BlockSpec((bn, din), lambda i: (i, 0)),
    ]
    args = [s0, s1, h]
    if not emit_cnt:
        in_specs.append(pl.BlockSpec((bn, 1), lambda i: (i, 0)))
        args.append(cnt)
    in_specs += [
        pl.BlockSpec((ci, co), lambda i: (0, 0)),
        pl.BlockSpec((1, co), lambda i: (0, 0)),
    ]
    args += [root, bias]
    if emit_cnt:
        out_specs = (pl.BlockSpec((bn, dout), lambda i: (i, 0)),
                     pl.BlockSpec((bn, 1), lambda i: (i, 0)))
        out_shape = (jax.ShapeDtypeStruct((N, dout), jnp.float32),
                     jax.ShapeDtypeStruct((N, 1), jnp.float32))
    else:
        out_specs = pl.BlockSpec((bn, dout), lambda i: (i, 0))
        out_shape = jax.ShapeDtypeStruct((N, dout), jnp.float32)
    return pl.pallas_call(
        body, grid=(grid,), in_specs=in_specs, out_specs=out_specs,
        out_shape=out_shape)(*args)


def _tc_pool_call(h4, ct, bt, wt, cb):
    def body(h_ref, ct_ref, bt_ref, wt_ref, cb_ref, out_ref):
        h = h_ref[...][:, :64]
        seg = jnp.where(ct_ref[...] == 1, bt_ref[...], -1)
        rows = jax.lax.broadcasted_iota(jnp.int32, (NUM_GRAPHS, N), 0)
        oh = (rows == seg).astype(jnp.float32)
        s = jnp.dot(oh, h, preferred_element_type=jnp.float32)
        cnt = jnp.sum(oh, axis=1, keepdims=True)
        pooled = s / jnp.maximum(cnt, 1.0)
        logits = jnp.dot(pooled, wt_ref[...],
                         preferred_element_type=jnp.float32) + cb_ref[...]
        out_ref[...] = 1.0 / (1.0 + jnp.exp(-logits))

    return pl.pallas_call(
        body,
        out_shape=jax.ShapeDtypeStruct((NUM_GRAPHS, 1), jnp.float32),
    )(h4, ct, bt, wt, cb)


def _relayout(p, ci, co):
    d = ci * co
    emb_r = p['emb'].reshape(NTYPES, ci, co).transpose(0, 2, 1).reshape(NTYPES, d)

    def pv(v):
        return v.reshape(ci, co).T.reshape(1, d)

    hb = jnp.concatenate([pv(p['bh']), pv(p['Wh'][:, 0]), pv(p['Wh'][:, 1])], 0)
    gb = jnp.concatenate([pv(p['bg']), pv(p['Wg'][:, 0]), pv(p['Wg'][:, 1])], 0)
    return emb_r, hb, gb


def kernel(x, edge_index, edge_attr, cell_type, batch, params):
    pad = EPAD - E
    src2d = jnp.concatenate(
        [edge_index[0], jnp.zeros((pad,), jnp.int32)]).reshape(EPAD // CH, CH)
    dst2d = jnp.concatenate(
        [edge_index[1], jnp.full((pad,), N, jnp.int32)]).reshape(EPAD // CH, CH)
    et = jnp.pad(edge_attr[:, 0:1], ((0, pad), (0, 0)))
    f0 = jnp.pad(edge_attr[:, 1:2], ((0, pad), (0, 0)))
    f1 = jnp.pad(edge_attr[:, 2:3], ((0, pad), (0, 0)))

    names = ['conv1', 'conv2', 'conv3', 'conv4']
    tabs = [_relayout(params[n], ci, co) for n, (ci, co) in zip(names, DIMS)]
    ts = _tc_prep_call(tabs)

    bes = [512, 2048, 2048, 512]
    h = x
    cnt = None
    zeros = jnp.zeros((NA // NS, DW), jnp.float32)
    for li, (name, (ci, co)) in enumerate(zip(names, DIMS)):
        p = params[name]
        dm = DW
        hj = _sc_gather_call(h, src2d, DW)
        msg = _tc_msg_call(hj, et, f0, f1, ts[li], ci, co, dm, bes[li],
                           count_col=(li == 0))
        part = _sc_scatter_call(msg, dst2d, zeros, dm)
        bias = p['bias'].reshape(1, co)
        if li == 0:
            h, cnt = _tc_combine_call(part[0], part[1], h, None, p['root'],
                                      bias, ci, co, DW, True)
        else:
            h = _tc_combine_call(part[0], part[1], h, cnt, p['root'], bias,
                                 ci, co, DW, False)

    ct = cell_type.reshape(1, N)
    bt = batch.reshape(1, N)
    return _tc_pool_call(h, ct, bt, params['cls_W'].T,
                         params['cls_b'].reshape(1, 1))


# speedup vs baseline: 2.2147x; 1.0008x over previous
import functools

import jax
import jax.numpy as jnp
from jax.experimental import pallas as pl
from jax.experimental.pallas import tpu as pltpu
from jax.experimental.pallas import tpu_sc as plsc

N = 10000
E = 160000
NUM_GRAPHS = 8
NTYPES = 36
DIMS = [(128, 8), (8, 8), (8, 8), (8, 64)]

NC = 2
NS = 16
NW = NC * NS
CH = 128
CW = 40
EW = CH * CW
EPAD = EW * NW
NA = 10240
DW = 128
GNB = 6
SNB = 2


def _sc_gather_call(table, src2d, dg):
    mesh = plsc.VectorSubcoreMesh(core_axis_name="c", subcore_axis_name="s")

    dt = table.dtype

    @functools.partial(
        pl.kernel,
        mesh=mesh,
        out_type=jax.ShapeDtypeStruct((EPAD, dg), dt),
        scratch_types=[
            pltpu.VMEM((CW, CH), jnp.int32),
            pltpu.VMEM((GNB, CH, dg), dt),
            pltpu.SemaphoreType.DMA,
            pltpu.SemaphoreType.DMA,
        ],
    )
    def gk(h_hbm, src_hbm, out_hbm, idx_v, buf_v, gsem, wsem):
        cid = jax.lax.axis_index("c")
        sid = jax.lax.axis_index("s")
        wid = cid * NS + sid
        pltpu.sync_copy(src_hbm.at[pl.ds(wid * CW, CW)], idx_v)

        def start_g(j, slot):
            return pltpu.async_copy(h_hbm.at[idx_v.at[j]], buf_v.at[slot], gsem)

        def start_w(j, slot):
            return pltpu.async_copy(
                buf_v.at[slot], out_hbm.at[pl.ds(wid * EW + j * CH, CH)], wsem)

        gh = {j: start_g(j, j % GNB) for j in range(min(GNB, CW))}
        wh = {}
        for j in range(CW):
            gh[j].wait()
            wh[j] = start_w(j, j % GNB)
            if j >= 2:
                wh[j - 2].wait()
                nxt = j - 2 + GNB
                if nxt < CW:
                    gh[nxt] = start_g(nxt, nxt % GNB)
        wh[CW - 2].wait()
        wh[CW - 1].wait()

    return gk(table, src2d)


def _sc_scatter_call(msg, dst2d, zeros, dm):
    mesh = plsc.VectorSubcoreMesh(core_axis_name="c", subcore_axis_name="s")
    zr = NA // NS

    @functools.partial(
        pl.kernel,
        mesh=mesh,
        out_type=jax.ShapeDtypeStruct((NC, NA, dm), jnp.float32),
        scratch_types=[
            pltpu.VMEM((CW, CH), jnp.int32),
            pltpu.VMEM((SNB, CH, dm), jnp.float32),
            pltpu.VMEM_SHARED((NA, dm), jnp.float32),
            pltpu.SemaphoreType.DMA,
            pltpu.SemaphoreType.DMA,
            pltpu.SemaphoreType.DMA,
        ],
    )
    def sk(msg_hbm, dst_hbm, z_hbm, out_hbm, idx_v, buf_v, acc_sh,
           lsem, asem, osem):
        cid = jax.lax.axis_index("c")
        sid = jax.lax.axis_index("s")
        wid = cid * NS + sid

        pltpu.sync_copy(dst_hbm.at[pl.ds(wid * CW, CW)], idx_v)
        pltpu.sync_copy(z_hbm, acc_sh.at[pl.ds(sid * zr, zr)])
        plsc.subcore_barrier()

        def start_l(j, slot):
            return pltpu.async_copy(
                msg_hbm.at[pl.ds(wid * EW + j * CH, CH)], buf_v.at[slot], lsem)

        def start_a(j, slot):
            return pltpu.async_copy(buf_v.at[slot], acc_sh.at[idx_v.at[j]],
                                    asem, add=True)

        lh = {0: start_l(0, 0)}
        ah = {}
        for j in range(CW):
            lh[j].wait()
            ah[j] = start_a(j, j % SNB)
            if j >= 1:
                ah[j - 1].wait()
            if j + 1 < CW:
                lh[j + 1] = start_l(j + 1, (j + 1) % SNB)
        ah[CW - 1].wait()
        plsc.subcore_barrier()

        dh = [pltpu.async_copy(acc_sh.at[pl.ds(sid * zr + r * CH, CH)],
                               out_hbm.at[cid, pl.ds(sid * zr + r * CH, CH)],
                               osem)
              for r in range(zr // CH)]
        for h in dh:
            h.wait()

    return sk(msg, dst2d, zeros)


def _tc_prep_call(tabs):
    def body(*refs):
        ins, outs = refs[:12], refs[12:]
        for li in range(4):
            e = ins[3 * li][...]
            hb = ins[3 * li + 1]
            gb = ins[3 * li + 2]
            rows = [e * hb[k:k + 1, :] + gb[k:k + 1, :] for k in range(3)]
            outs[li][...] = jnp.concatenate(rows, axis=0)

    flat = [a for t in tabs for a in t]
    out_shape = tuple(
        jax.ShapeDtypeStruct((108, t[0].shape[1]), jnp.float32) for t in tabs)
    return pl.pallas_call(body, out_shape=out_shape)(*flat)


def _tc_msg_call(hj, et, f0, f1, tab, ci, co, dm, be, count_col):
    d = ci * co
    dgin = hj.shape[1]
    grid = EPAD // be

    def body(hj_ref, et_ref, f0_ref, f1_ref, t_ref, out_ref):
        lane = jax.lax.broadcasted_iota(jnp.int32, (be, 3 * NTYPES), 1)
        lt = lane - NTYPES * (lane // NTYPES)
        e = et_ref[...].astype(jnp.int32)
        m = lt == e
        coeff = jnp.where(lane < NTYPES, 1.0,
                          jnp.where(lane < 2 * NTYPES, f0_ref[...], f1_ref[...]))
        p = jnp.where(m, coeff, 0.0)
        arg = jnp.dot(p.astype(jnp.bfloat16), t_ref[...].astype(jnp.bfloat16),
                      preferred_element_type=jnp.float32)
        w = jnp.maximum(arg, 0.0)
        hjc = hj_ref[...][:, :ci]
        ht = jnp.concatenate([hjc] * co, axis=1).astype(jnp.float32)
        prod = w * ht
        ko = jax.lax.broadcasted_iota(jnp.int32, (d, co), 0) // ci
        oo = jax.lax.broadcasted_iota(jnp.int32, (d, co), 1)
        red = (ko == oo).astype(jnp.bfloat16)
        msg = jnp.dot(prod.astype(jnp.bfloat16), red,
                      preferred_element_type=jnp.float32)
        if dm > co:
            cols = [msg]
            if count_col:
                cols.append(jnp.ones((be, 1), jnp.float32))
                cols.append(jnp.zeros((be, dm - co - 1), jnp.float32))
            else:
                cols.append(jnp.zeros((be, dm - co), jnp.float32))
            out_ref[...] = jnp.concatenate(cols, axis=1)
        else:
            out_ref[...] = msg

    return pl.pallas_call(
        body,
        grid=(grid,),
        in_specs=[
            pl.BlockSpec((be, dgin), lambda i: (i, 0)),
            pl.BlockSpec((be, 1), lambda i: (i, 0)),
            pl.BlockSpec((be, 1), lambda i: (i, 0)),
            pl.BlockSpec((be, 1), lambda i: (i, 0)),
            pl.BlockSpec((108, d), lambda i: (0, 0)),
        ],
        out_specs=pl.BlockSpec((be, dm), lambda i: (i, 0)),
        out_shape=jax.ShapeDtypeStruct((EPAD, dm), jnp.float32),
    )(hj, et, f0, f1, tab)


def _tc_combine_call(s0, s1, h, cnt, root, bias, ci, co, dout, emit_cnt):
    bn = 1000
    grid = N // bn
    dm = s0.shape[1]
    din = h.shape[1]

    def body(*refs):
        if emit_cnt:
            s0_ref, s1_ref, h_ref, root_ref, bias_ref, out_ref, cnt_ref = refs
        else:
            s0_ref, s1_ref, h_ref, cin_ref, root_ref, bias_ref, out_ref = refs
        p0 = s0_ref[...].astype(jnp.float32)
        p1 = s1_ref[...].astype(jnp.float32)
        s = p0[:, :co] + p1[:, :co]
        if emit_cnt:
            c = p0[:, co:co + 1] + p1[:, co:co + 1]
        else:
            c = cin_ref[...]
        agg = s / jnp.maximum(c, 1.0)
        hc = h_ref[...][:, :ci].astype(jnp.float32)
        o = jnp.maximum(
            agg + jnp.dot(hc, root_ref[...], preferred_element_type=jnp.float32)
            + bias_ref[...], 0.0)
        if dout > co:
            o = jnp.concatenate(
                [o, jnp.zeros((bn, dout - co), jnp.float32)], axis=1)
        out_ref[...] = o
        if emit_cnt:
            cnt_ref[...] = c

    in_specs = [
        pl.BlockSpec((bn, dm), lambda i: (i, 0)),
        pl.BlockSpec((bn, dm), lambda i: (i, 0)),
        pl.BlockSpec((bn, din), lambda i: (i, 0)),
    ]
    args = [s0, s1, h]
    if not emit_cnt:
        in_specs.append(pl.BlockSpec((bn, 1), lambda i: (i, 0)))
        args.append(cnt)
    in_specs += [
        pl.BlockSpec((ci, co), lambda i: (0, 0)),
        pl.BlockSpec((1, co), lambda i: (0, 0)),
    ]
    args += [root, bias]
    if emit_cnt:
        out_specs = (pl.BlockSpec((bn, dout), lambda i: (i, 0)),
                     pl.BlockSpec((bn, 1), lambda i: (i, 0)))
        out_shape = (jax.ShapeDtypeStruct((N, dout), jnp.float32),
                     jax.ShapeDtypeStruct((N, 1), jnp.float32))
    else:
        out_specs = pl.BlockSpec((bn, dout), lambda i: (i, 0))
        out_shape = jax.ShapeDtypeStruct((N, dout), jnp.float32)
    return pl.pallas_call(
        body, grid=(grid,), in_specs=in_specs, out_specs=out_specs,
        out_shape=out_shape)(*args)


def _tc_pool_call(h4, ct, bt, wt, cb):
    def body(h_ref, ct_ref, bt_ref, wt_ref, cb_ref, out_ref):
        h = h_ref[...][:, :64].astype(jnp.float32)
        seg = jnp.where(ct_ref[...] == 1, bt_ref[...], -1)
        rows = jax.lax.broadcasted_iota(jnp.int32, (NUM_GRAPHS, N), 0)
        oh = (rows == seg).astype(jnp.float32)
        s = jnp.dot(oh, h, preferred_element_type=jnp.float32)
        cnt = jnp.sum(oh, axis=1, keepdims=True)
        pooled = s / jnp.maximum(cnt, 1.0)
        logits = jnp.dot(pooled, wt_ref[...],
                         preferred_element_type=jnp.float32) + cb_ref[...]
        out_ref[...] = 1.0 / (1.0 + jnp.exp(-logits))

    return pl.pallas_call(
        body,
        out_shape=jax.ShapeDtypeStruct((NUM_GRAPHS, 1), jnp.float32),
    )(h4, ct, bt, wt, cb)


def _relayout(p, ci, co):
    d = ci * co
    emb_r = p['emb'].reshape(NTYPES, ci, co).transpose(0, 2, 1).reshape(NTYPES, d)

    def pv(v):
        return v.reshape(ci, co).T.reshape(1, d)

    hb = jnp.concatenate([pv(p['bh']), pv(p['Wh'][:, 0]), pv(p['Wh'][:, 1])], 0)
    gb = jnp.concatenate([pv(p['bg']), pv(p['Wg'][:, 0]), pv(p['Wg'][:, 1])], 0)
    return emb_r, hb, gb


def kernel(x, edge_index, edge_attr, cell_type, batch, params):
    pad = EPAD - E
    src2d = jnp.concatenate(
        [edge_index[0], jnp.zeros((pad,), jnp.int32)]).reshape(EPAD // CH, CH)
    dst2d = jnp.concatenate(
        [edge_index[1], jnp.full((pad,), N, jnp.int32)]).reshape(EPAD // CH, CH)
    et = jnp.pad(edge_attr[:, 0:1], ((0, pad), (0, 0)))
    f0 = jnp.pad(edge_attr[:, 1:2], ((0, pad), (0, 0)))
    f1 = jnp.pad(edge_attr[:, 2:3], ((0, pad), (0, 0)))

    names = ['conv1', 'conv2', 'conv3', 'conv4']
    tabs = [_relayout(params[n], ci, co) for n, (ci, co) in zip(names, DIMS)]
    ts = _tc_prep_call(tabs)

    bes = [512, 2048, 2048, 512]
    h = x
    cnt = None
    zeros = jnp.zeros((NA // NS, DW), jnp.float32)
    for li, (name, (ci, co)) in enumerate(zip(names, DIMS)):
        p = params[name]
        dm = DW
        hj = _sc_gather_call(h, src2d, DW)
        msg = _tc_msg_call(hj, et, f0, f1, ts[li], ci, co, dm, bes[li],
                           count_col=(li == 0))
        part = _sc_scatter_call(msg, dst2d, zeros, dm)
        bias = p['bias'].reshape(1, co)
        if li == 0:
            h, cnt = _tc_combine_call(part[0], part[1], h, None, p['root'],
                                      bias, ci, co, DW, True)
        else:
            h = _tc_combine_call(part[0], part[1], h, cnt, p['root'], bias,
                                 ci, co, DW, False)

    ct = cell_type.reshape(1, N)
    bt = batch.reshape(1, N)
    return _tc_pool_call(h, ct, bt, params['cls_W'].T,
                         params['cls_b'].reshape(1, 1))


# speedup vs baseline: 2.3929x; 1.0805x over previous
import functools

import jax
import jax.numpy as jnp
from jax.experimental import pallas as pl
from jax.experimental.pallas import tpu as pltpu
from jax.experimental.pallas import tpu_sc as plsc

N = 10000
E = 160000
NUM_GRAPHS = 8
NTYPES = 36
DIMS = [(128, 8), (8, 8), (8, 8), (8, 64)]

NC = 2
NS = 16
NW = NC * NS
CH = 128
CW = 40
EW = CH * CW
EPAD = EW * NW
NA = 10240
DW = 128
GNB = 6
SNB = 2


def _sc_gather_call(table, src2d, dg):
    mesh = plsc.VectorSubcoreMesh(core_axis_name="c", subcore_axis_name="s")

    dt = table.dtype

    @functools.partial(
        pl.kernel,
        mesh=mesh,
        out_type=jax.ShapeDtypeStruct((EPAD, dg), dt),
        scratch_types=[
            pltpu.VMEM((CW, CH), jnp.int32),
            pltpu.VMEM((GNB, CH, dg), dt),
            pltpu.SemaphoreType.DMA,
            pltpu.SemaphoreType.DMA,
        ],
    )
    def gk(h_hbm, src_hbm, out_hbm, idx_v, buf_v, gsem, wsem):
        cid = jax.lax.axis_index("c")
        sid = jax.lax.axis_index("s")
        wid = cid * NS + sid
        pltpu.sync_copy(src_hbm.at[pl.ds(wid * CW, CW)], idx_v)

        def start_g(j, slot):
            return pltpu.async_copy(h_hbm.at[idx_v.at[j]], buf_v.at[slot], gsem)

        def start_w(j, slot):
            return pltpu.async_copy(
                buf_v.at[slot], out_hbm.at[pl.ds(wid * EW + j * CH, CH)], wsem)

        gh = {j: start_g(j, j % GNB) for j in range(min(GNB, CW))}
        wh = {}
        for j in range(CW):
            gh[j].wait()
            wh[j] = start_w(j, j % GNB)
            if j >= 2:
                wh[j - 2].wait()
                nxt = j - 2 + GNB
                if nxt < CW:
                    gh[nxt] = start_g(nxt, nxt % GNB)
        wh[CW - 2].wait()
        wh[CW - 1].wait()

    return gk(table, src2d)


def _sc_scatter_call(msg, dst2d, zeros, dm):
    mesh = plsc.VectorSubcoreMesh(core_axis_name="c", subcore_axis_name="s")
    zr = NA // NS

    @functools.partial(
        pl.kernel,
        mesh=mesh,
        out_type=jax.ShapeDtypeStruct((NC, NA, dm), jnp.float32),
        scratch_types=[
            pltpu.VMEM((CW, CH), jnp.int32),
            pltpu.VMEM((SNB, CH, dm), jnp.float32),
            pltpu.VMEM_SHARED((NA, dm), jnp.float32),
            pltpu.SemaphoreType.DMA,
            pltpu.SemaphoreType.DMA,
            pltpu.SemaphoreType.DMA,
        ],
    )
    def sk(msg_hbm, dst_hbm, z_hbm, out_hbm, idx_v, buf_v, acc_sh,
           lsem, asem, osem):
        cid = jax.lax.axis_index("c")
        sid = jax.lax.axis_index("s")
        wid = cid * NS + sid

        pltpu.sync_copy(dst_hbm.at[pl.ds(wid * CW, CW)], idx_v)
        pltpu.sync_copy(z_hbm, acc_sh.at[pl.ds(sid * zr, zr)])
        plsc.subcore_barrier()

        def start_l(j, slot):
            return pltpu.async_copy(
                msg_hbm.at[pl.ds(wid * EW + j * CH, CH)], buf_v.at[slot], lsem)

        def start_a(j, slot):
            return pltpu.async_copy(buf_v.at[slot], acc_sh.at[idx_v.at[j]],
                                    asem, add=True)

        lh = {0: start_l(0, 0)}
        ah = {}
        for j in range(CW):
            lh[j].wait()
            ah[j] = start_a(j, j % SNB)
            if j >= 1:
                ah[j - 1].wait()
            if j + 1 < CW:
                lh[j + 1] = start_l(j + 1, (j + 1) % SNB)
        ah[CW - 1].wait()
        plsc.subcore_barrier()

        dh = [pltpu.async_copy(acc_sh.at[pl.ds(sid * zr + r * CH, CH)],
                               out_hbm.at[cid, pl.ds(sid * zr + r * CH, CH)],
                               osem)
              for r in range(zr // CH)]
        for h in dh:
            h.wait()

    return sk(msg, dst2d, zeros)


def _tc_prep_call(tabs):
    def body(*refs):
        ins, outs = refs[:12], refs[12:]
        for li in range(4):
            e = ins[3 * li][...]
            hb = ins[3 * li + 1]
            gb = ins[3 * li + 2]
            rows = [e * hb[k:k + 1, :] + gb[k:k + 1, :] for k in range(3)]
            outs[li][...] = jnp.concatenate(rows, axis=0)

    flat = [a for t in tabs for a in t]
    out_shape = tuple(
        jax.ShapeDtypeStruct((108, t[0].shape[1]), jnp.float32) for t in tabs)
    return pl.pallas_call(body, out_shape=out_shape)(*flat)


def _tc_msg_call(hj, et, f0, f1, tab, ci, co, dm, be, count_col):
    d = ci * co
    dgin = hj.shape[1]
    grid = EPAD // be

    def body(hj_ref, et_ref, f0_ref, f1_ref, t_ref, out_ref):
        lane = jax.lax.broadcasted_iota(jnp.int32, (be, 3 * NTYPES), 1)
        lt = lane - NTYPES * (lane // NTYPES)
        e = et_ref[...].astype(jnp.int32)
        m = lt == e
        coeff = jnp.where(lane < NTYPES, 1.0,
                          jnp.where(lane < 2 * NTYPES, f0_ref[...], f1_ref[...]))
        p = jnp.where(m, coeff, 0.0)
        arg = jnp.dot(p.astype(jnp.bfloat16), t_ref[...].astype(jnp.bfloat16),
                      preferred_element_type=jnp.float32)
        w = jnp.maximum(arg, 0.0)
        hjc = hj_ref[...][:, :ci]
        ht = jnp.concatenate([hjc] * co, axis=1).astype(jnp.float32)
        prod = w * ht
        ko = jax.lax.broadcasted_iota(jnp.int32, (d, co), 0) // ci
        oo = jax.lax.broadcasted_iota(jnp.int32, (d, co), 1)
        red = (ko == oo).astype(jnp.bfloat16)
        msg = jnp.dot(prod.astype(jnp.bfloat16), red,
                      preferred_element_type=jnp.float32)
        if dm > co:
            cols = [msg]
            if count_col:
                cols.append(jnp.ones((be, 1), jnp.float32))
                cols.append(jnp.zeros((be, dm - co - 1), jnp.float32))
            else:
                cols.append(jnp.zeros((be, dm - co), jnp.float32))
            out_ref[...] = jnp.concatenate(cols, axis=1)
        else:
            out_ref[...] = msg

    return pl.pallas_call(
        body,
        grid=(grid,),
        in_specs=[
            pl.BlockSpec((be, dgin), lambda i: (i, 0)),
            pl.BlockSpec((be, 1), lambda i: (i, 0)),
            pl.BlockSpec((be, 1), lambda i: (i, 0)),
            pl.BlockSpec((be, 1), lambda i: (i, 0)),
            pl.BlockSpec((108, d), lambda i: (0, 0)),
        ],
        out_specs=pl.BlockSpec((be, dm), lambda i: (i, 0)),
        out_shape=jax.ShapeDtypeStruct((EPAD, dm), jnp.float32),
    )(hj, et, f0, f1, tab)


def _tc_combine_call(part, h, cnt, root, bias, ci, co, dout, emit_cnt):
    bn = 1000
    grid = N // bn
    dm = part.shape[2]
    din = h.shape[1]

    def body(*refs):
        if emit_cnt:
            s0_ref, s1_ref, h_ref, root_ref, bias_ref, out_ref, cnt_ref = refs
        else:
            s0_ref, s1_ref, h_ref, cin_ref, root_ref, bias_ref, out_ref = refs
        p0 = s0_ref[...][0].astype(jnp.float32)
        p1 = s1_ref[...][0].astype(jnp.float32)
        s = p0[:, :co] + p1[:, :co]
        if emit_cnt:
            c = p0[:, co:co + 1] + p1[:, co:co + 1]
        else:
            c = cin_ref[...]
        agg = s / jnp.maximum(c, 1.0)
        hc = h_ref[...][:, :ci].astype(jnp.float32)
        o = jnp.maximum(
            agg + jnp.dot(hc, root_ref[...], preferred_element_type=jnp.float32)
            + bias_ref[...], 0.0)
        if dout > co:
            o = jnp.concatenate(
                [o, jnp.zeros((bn, dout - co), jnp.float32)], axis=1)
        out_ref[...] = o
        if emit_cnt:
            cnt_ref[...] = c

    in_specs = [
        pl.BlockSpec((1, bn, dm), lambda i: (0, i, 0)),
        pl.BlockSpec((1, bn, dm), lambda i: (1, i, 0)),
        pl.BlockSpec((bn, din), lambda i: (i, 0)),
    ]
    args = [part, part, h]
    if not emit_cnt:
        in_specs.append(pl.BlockSpec((bn, 1), lambda i: (i, 0)))
        args.append(cnt)
    in_specs += [
        pl.BlockSpec((ci, co), lambda i: (0, 0)),
        pl.BlockSpec((1, co), lambda i: (0, 0)),
    ]
    args += [root, bias]
    if emit_cnt:
        out_specs = (pl.BlockSpec((bn, dout), lambda i: (i, 0)),
                     pl.BlockSpec((bn, 1), lambda i: (i, 0)))
        out_shape = (jax.ShapeDtypeStruct((N, dout), jnp.float32),
                     jax.ShapeDtypeStruct((N, 1), jnp.float32))
    else:
        out_specs = pl.BlockSpec((bn, dout), lambda i: (i, 0))
        out_shape = jax.ShapeDtypeStruct((N, dout), jnp.float32)
    return pl.pallas_call(
        body, grid=(grid,), in_specs=in_specs, out_specs=out_specs,
        out_shape=out_shape)(*args)


def _tc_pool_call(h4, ct, bt, wt, cb):
    def body(h_ref, ct_ref, bt_ref, wt_ref, cb_ref, out_ref):
        h = h_ref[...][:, :64].astype(jnp.float32)
        seg = jnp.where(ct_ref[...] == 1, bt_ref[...], -1)
        rows = jax.lax.broadcasted_iota(jnp.int32, (NUM_GRAPHS, N), 0)
        oh = (rows == seg).astype(jnp.float32)
        s = jnp.dot(oh, h, preferred_element_type=jnp.float32)
        cnt = jnp.sum(oh, axis=1, keepdims=True)
        pooled = s / jnp.maximum(cnt, 1.0)
        logits = jnp.dot(pooled, wt_ref[...],
                         preferred_element_type=jnp.float32) + cb_ref[...]
        out_ref[...] = 1.0 / (1.0 + jnp.exp(-logits))

    return pl.pallas_call(
        body,
        out_shape=jax.ShapeDtypeStruct((NUM_GRAPHS, 1), jnp.float32),
    )(h4, ct, bt, wt, cb)


def _relayout(p, ci, co):
    d = ci * co
    emb_r = p['emb'].reshape(NTYPES, ci, co).transpose(0, 2, 1).reshape(NTYPES, d)

    def pv(v):
        return v.reshape(ci, co).T.reshape(1, d)

    hb = jnp.concatenate([pv(p['bh']), pv(p['Wh'][:, 0]), pv(p['Wh'][:, 1])], 0)
    gb = jnp.concatenate([pv(p['bg']), pv(p['Wg'][:, 0]), pv(p['Wg'][:, 1])], 0)
    return emb_r, hb, gb


def kernel(x, edge_index, edge_attr, cell_type, batch, params):
    pad = EPAD - E
    src2d = jnp.concatenate(
        [edge_index[0], jnp.zeros((pad,), jnp.int32)]).reshape(EPAD // CH, CH)
    dst2d = jnp.concatenate(
        [edge_index[1], jnp.full((pad,), N, jnp.int32)]).reshape(EPAD // CH, CH)
    et = jnp.pad(edge_attr[:, 0:1], ((0, pad), (0, 0)))
    f0 = jnp.pad(edge_attr[:, 1:2], ((0, pad), (0, 0)))
    f1 = jnp.pad(edge_attr[:, 2:3], ((0, pad), (0, 0)))

    names = ['conv1', 'conv2', 'conv3', 'conv4']
    tabs = [_relayout(params[n], ci, co) for n, (ci, co) in zip(names, DIMS)]
    ts = _tc_prep_call(tabs)

    bes = [1024, 2048, 2048, 1024]
    h = x
    cnt = None
    zeros = jnp.zeros((NA // NS, DW), jnp.float32)
    for li, (name, (ci, co)) in enumerate(zip(names, DIMS)):
        p = params[name]
        dm = DW
        hj = _sc_gather_call(h, src2d, DW)
        msg = _tc_msg_call(hj, et, f0, f1, ts[li], ci, co, dm, bes[li],
                           count_col=(li == 0))
        part = _sc_scatter_call(msg, dst2d, zeros, dm)
        bias = p['bias'].reshape(1, co)
        if li == 0:
            h, cnt = _tc_combine_call(part, h, None, p['root'],
                                      bias, ci, co, DW, True)
        else:
            h = _tc_combine_call(part, h, cnt, p['root'], bias,
                                 ci, co, DW, False)

    ct = cell_type.reshape(1, N)
    bt = batch.reshape(1, N)
    return _tc_pool_call(h, ct, bt, params['cls_W'].T,
                         params['cls_b'].reshape(1, 1))
